# baseline (speedup 1.0000x reference)
import functools

import jax
import jax.numpy as jnp
from jax import lax
from jax.experimental import pallas as pl
from jax.experimental.pallas import tpu as pltpu
from jax.experimental.pallas import tpu_sc as plsc

NC = 2
NS = 16
L = 16

P = 20
C = 10
ACC = 384
PCI_PAD = 224
CHUNK = 2048


def _sc_body(pix_per_w, chunks, pa_hbm, lab_hbm, pci_hbm, out_hbm,
             dbuf, lbuf, accs, accc, rbuf0, rbuf1, pcib, outv, cstage,
             shared):
    c = lax.axis_index("c")
    s = lax.axis_index("s")
    w = c * NS + s
    iota = lax.iota(jnp.int32, L)
    zf = jnp.zeros((L,), jnp.float32)
    ones = jnp.ones((L,), jnp.float32)

    for j in range(ACC // L):
        accs[pl.ds(j * L, L)] = zf
        accc[pl.ds(j * L, L)] = zf

    pps = [(iota + 16 * i) // P for i in range(5)]
    prs = [(iota + 16 * i) % P for i in range(5)]

    for k in range(chunks):
        base_pix = w * pix_per_w + k * CHUNK
        pltpu.sync_copy(pa_hbm.at[pl.ds(base_pix * P, CHUNK * P)], dbuf)
        pltpu.sync_copy(lab_hbm.at[pl.ds(base_pix, CHUNK)], lbuf)

        def group(g, carry):
            for i in range(5):
                off = g * 80 + i * 16
                x = jnp.abs(dbuf[pl.ds(off, 16)])
                lbl = plsc.load_gather(lbuf, [pps[i] + g * 4])
                dest = lbl * P + prs[i]
                plsc.addupdate_scatter(accs, [dest], x)
                plsc.addupdate_scatter(accc, [dest], ones)
            return carry

        lax.fori_loop(0, CHUNK * P // 80, group, 0)

    pltpu.sync_copy(accs, shared.at[s, pl.ds(0, ACC)])
    cstage[pl.ds(0, L)] = plsc.load_gather(accc, [iota * P])
    pltpu.sync_copy(cstage, shared.at[s, pl.ds(ACC, L)])
    plsc.subcore_barrier()

    @pl.when(s == 0)
    def _finalize():
        pltpu.sync_copy(pci_hbm, pcib)
        lane_ok = iota < C
        ls = zf
        tot = zf
        for bl in range(NS // 2):
            pltpu.sync_copy(shared.at[2 * bl], rbuf0)
            pltpu.sync_copy(shared.at[2 * bl + 1], rbuf1)
            cnt = (plsc.load_gather(rbuf0, [ACC + 1 + iota]) +
                   plsc.load_gather(rbuf1, [ACC + 1 + iota]))
            t = zf
            npv = zf
            for p in range(P):
                m_raw = plsc.load_gather(pcib, [p * C + iota])
                m = jnp.where(m_raw != 0, 1.0, 0.0).astype(jnp.float32)
                sp = (plsc.load_gather(rbuf0, [(iota + 1) * P + p]) +
                      plsc.load_gather(rbuf1, [(iota + 1) * P + p]))
                t = t + m * sp
                npv = npv + m
            valid = (cnt > 0.0) & (npv > 0.0) & lane_ok
            denom = jnp.where(valid, cnt * npv, 1.0)
            ls = ls + jnp.where(valid, t / denom, 0.0)
            tot = tot + jnp.where(valid, 1.0, 0.0)
        outv[pl.ds(0, L)] = ls
        outv[pl.ds(L, L)] = tot
        pltpu.sync_copy(outv, out_hbm.at[c])


def kernel(prototype_activations, target_labels, prototype_class_identity):
    B, N, Pp = prototype_activations.shape
    assert Pp == P and prototype_class_identity.shape == (P, C)
    n_pix = B * N
    pix_per_w = n_pix // (NC * NS)
    chunks = pix_per_w // CHUNK

    pa_flat = prototype_activations.reshape(-1)
    lab_flat = target_labels.reshape(-1).astype(jnp.int32)
    pci_pad = jnp.zeros((PCI_PAD,), jnp.float32).at[: P * C].set(
        prototype_class_identity.reshape(-1).astype(jnp.float32))

    mesh = plsc.VectorSubcoreMesh(
        core_axis_name="c", subcore_axis_name="s",
        num_cores=NC, num_subcores=NS)

    run = functools.partial(
        pl.kernel,
        out_type=jax.ShapeDtypeStruct((NC, 2 * L), jnp.float32),
        mesh=mesh,
        scratch_types=[
            pltpu.VMEM((CHUNK * P,), jnp.float32),
            pltpu.VMEM((CHUNK,), jnp.int32),
            pltpu.VMEM((ACC,), jnp.float32),
            pltpu.VMEM((ACC,), jnp.float32),
            pltpu.VMEM((512,), jnp.float32),
            pltpu.VMEM((512,), jnp.float32),
            pltpu.VMEM((PCI_PAD,), jnp.float32),
            pltpu.VMEM((2 * L,), jnp.float32),
            pltpu.VMEM((L,), jnp.float32),
            pltpu.VMEM_SHARED((NS, 512), jnp.float32),
        ],
        compiler_params=pltpu.CompilerParams(needs_layout_passes=False),
    )(functools.partial(_sc_body, pix_per_w, chunks))

    out = run(pa_flat, lab_flat, pci_pad)
    ls = jnp.sum(out[:, :L])
    tot = jnp.sum(out[:, L:])
    return jnp.where(tot > 0, ls / jnp.where(tot > 0, tot, 1.0),
                     jnp.asarray(0.0, dtype=jnp.float32))


# speedup vs baseline: 1.7431x; 1.7431x over previous
import functools

import jax
import jax.numpy as jnp
from jax import lax
from jax.experimental import pallas as pl
from jax.experimental.pallas import tpu as pltpu
from jax.experimental.pallas import tpu_sc as plsc

NC = 2
NS = 16
L = 16

P = 20
C = 10
ACC = 384
PCI_PAD = 224
CHUNK = 2048


def _sc_body(pix_per_w, chunks, pa_hbm, lab_hbm, pci_hbm, out_hbm,
             dbuf0, dbuf1, lbuf0, lbuf1, accs, accc, rbuf0, rbuf1, pcib,
             outv, cstage, sem_d0, sem_l0, sem_d1, sem_l1, shared):
    c = lax.axis_index("c")
    s = lax.axis_index("s")
    w = c * NS + s
    iota = lax.iota(jnp.int32, L)
    zf = jnp.zeros((L,), jnp.float32)
    ones = jnp.ones((L,), jnp.float32)

    for j in range(ACC // L):
        accs[pl.ds(j * L, L)] = zf
        accc[pl.ds(j * L, L)] = zf

    pps = [(iota + 16 * i) // P for i in range(5)]
    prs = [(iota + 16 * i) % P for i in range(5)]

    bufs = [(dbuf0, lbuf0, sem_d0, sem_l0), (dbuf1, lbuf1, sem_d1, sem_l1)]

    def start(k, db, lb, sd, sl):
        bp = w * pix_per_w + k * CHUNK
        cd = pltpu.async_copy(pa_hbm.at[pl.ds(bp, CHUNK)], db, sd)
        cl = pltpu.async_copy(lab_hbm.at[pl.ds(bp, CHUNK)], lb, sl)
        return cd, cl

    pend = start(0, *bufs[0])
    for k in range(chunks):
        db, lb, _, _ = bufs[k % 2]
        nxt = start(k + 1, *bufs[(k + 1) % 2]) if k + 1 < chunks else None
        pend[0].wait()
        pend[1].wait()

        @functools.partial(plsc.parallel_loop, 0, CHUNK * P // 80, unroll=4)
        def group(g):
            for i in range(5):
                pix = pps[i] + g * 4
                x = plsc.load_gather(db, [pix, prs[i]])
                lbl = plsc.load_gather(lb, [pix])
                dest = lbl * P + prs[i]
                plsc.addupdate_scatter(accs, [dest], jnp.abs(x))
                plsc.addupdate_scatter(accc, [dest], ones)

        pend = nxt

    pltpu.sync_copy(accs, shared.at[s, pl.ds(0, ACC)])
    cstage[pl.ds(0, L)] = plsc.load_gather(accc, [iota * P])
    pltpu.sync_copy(cstage, shared.at[s, pl.ds(ACC, L)])
    plsc.subcore_barrier()

    @pl.when(s == 0)
    def _finalize():
        pltpu.sync_copy(pci_hbm, pcib)
        lane_ok = iota < C
        ls = zf
        tot = zf
        for bl in range(NS // 2):
            pltpu.sync_copy(shared.at[2 * bl], rbuf0)
            pltpu.sync_copy(shared.at[2 * bl + 1], rbuf1)
            cnt = (plsc.load_gather(rbuf0, [ACC + 1 + iota]) +
                   plsc.load_gather(rbuf1, [ACC + 1 + iota]))
            t = zf
            npv = zf
            for p in range(P):
                m_raw = plsc.load_gather(pcib, [p * C + iota])
                m = jnp.where(m_raw != 0, 1.0, 0.0).astype(jnp.float32)
                sp = (plsc.load_gather(rbuf0, [(iota + 1) * P + p]) +
                      plsc.load_gather(rbuf1, [(iota + 1) * P + p]))
                t = t + m * sp
                npv = npv + m
            valid = (cnt > 0.0) & (npv > 0.0) & lane_ok
            denom = jnp.where(valid, cnt * npv, 1.0)
            ls = ls + jnp.where(valid, t / denom, 0.0)
            tot = tot + jnp.where(valid, 1.0, 0.0)
        outv[pl.ds(0, L)] = ls
        outv[pl.ds(L, L)] = tot
        pltpu.sync_copy(outv, out_hbm.at[c])


def _tc_block(pa_ref, lab_ref, out_ref):
    nb = pl.program_id(1)
    lbl = lab_ref[0, 0, :]
    cls = jax.lax.broadcasted_iota(jnp.int32, (lbl.shape[0], C), 1) + 1
    onehot = jnp.where(lbl[:, None] == cls, 1.0, 0.0).astype(jnp.float32)
    x = jnp.abs(pa_ref[0])
    xx = jnp.concatenate([x, jnp.ones((x.shape[0], 1), jnp.float32)], axis=1)
    partial = jax.lax.dot_general(
        onehot, xx, (((0,), (0,)), ((), ())),
        preferred_element_type=jnp.float32)

    @pl.when(nb == 0)
    def _():
        out_ref[0] = jnp.zeros_like(out_ref[0])

    out_ref[0] += partial


def _tc_sums(pa, labels, block_n):
    B, N, _ = pa.shape
    lab3 = labels.reshape(B, 1, N)
    return pl.pallas_call(
        _tc_block,
        grid=(B, N // block_n),
        in_specs=[
            pl.BlockSpec((1, block_n, P), lambda b, n: (b, n, 0)),
            pl.BlockSpec((1, 1, block_n), lambda b, n: (b, 0, n)),
        ],
        out_specs=pl.BlockSpec((1, C, P + 1), lambda b, n: (b, 0, 0)),
        out_shape=jax.ShapeDtypeStruct((B, C, P + 1), jnp.float32),
        compiler_params=pltpu.CompilerParams(
            dimension_semantics=("parallel", "arbitrary")),
    )(pa, lab3)


def kernel(prototype_activations, target_labels, prototype_class_identity):
    B, N, Pp = prototype_activations.shape
    assert Pp == P and prototype_class_identity.shape == (P, C)
    out = _tc_sums(prototype_activations,
                   target_labels.astype(jnp.int32), 4096)
    sums = out[:, :, :P]
    counts = out[:, :, P]
    proto_mask = (prototype_class_identity != 0).astype(jnp.float32)
    n_protos = jnp.sum(proto_mask, axis=0)
    safe_counts = jnp.where(counts > 0, counts, 1.0)
    terms = jnp.einsum("bcp,pc->bc", sums / safe_counts[:, :, None],
                       proto_mask) / jnp.where(n_protos > 0, n_protos, 1.0)
    valid = ((counts > 0) & (n_protos[None, :] > 0)).astype(jnp.float32)
    total = jnp.sum(valid)
    loss_sum = jnp.sum(valid * terms)
    return jnp.where(total > 0,
                     loss_sum / jnp.where(total > 0, total, 1.0),
                     jnp.asarray(0.0, dtype=jnp.float32))


def _sc_kernel_unused(prototype_activations, target_labels, prototype_class_identity):
    B, N, Pp = prototype_activations.shape
    assert Pp == P and prototype_class_identity.shape == (P, C)
    n_pix = B * N
    pix_per_w = n_pix // (NC * NS)
    chunks = pix_per_w // CHUNK

    pa_flat = prototype_activations.reshape(n_pix, P)
    lab_flat = target_labels.reshape(-1).astype(jnp.int32)
    pci_pad = jnp.zeros((PCI_PAD,), jnp.float32).at[: P * C].set(
        prototype_class_identity.reshape(-1).astype(jnp.float32))

    mesh = plsc.VectorSubcoreMesh(
        core_axis_name="c", subcore_axis_name="s",
        num_cores=NC, num_subcores=NS)

    run = functools.partial(
        pl.kernel,
        out_type=jax.ShapeDtypeStruct((NC, 2 * L), jnp.float32),
        mesh=mesh,
        scratch_types=[
            pltpu.VMEM((CHUNK, P), jnp.float32),
            pltpu.VMEM((CHUNK, P), jnp.float32),
            pltpu.VMEM((CHUNK,), jnp.int32),
            pltpu.VMEM((CHUNK,), jnp.int32),
            pltpu.VMEM((ACC,), jnp.float32),
            pltpu.VMEM((ACC,), jnp.float32),
            pltpu.VMEM((512,), jnp.float32),
            pltpu.VMEM((512,), jnp.float32),
            pltpu.VMEM((PCI_PAD,), jnp.float32),
            pltpu.VMEM((2 * L,), jnp.float32),
            pltpu.VMEM((L,), jnp.float32),
            pltpu.SemaphoreType.DMA,
            pltpu.SemaphoreType.DMA,
            pltpu.SemaphoreType.DMA,
            pltpu.SemaphoreType.DMA,
            pltpu.VMEM_SHARED((NS, 512), jnp.float32),
        ],
        compiler_params=pltpu.CompilerParams(needs_layout_passes=False),
    )(functools.partial(_sc_body, pix_per_w, chunks))

    out = run(pa_flat, lab_flat, pci_pad)
    ls = jnp.sum(out[:, :L])
    tot = jnp.sum(out[:, L:])
    return jnp.where(tot > 0, ls / jnp.where(tot > 0, tot, 1.0),
                     jnp.asarray(0.0, dtype=jnp.float32))


# speedup vs baseline: 1.9894x; 1.1413x over previous
import functools

import jax
import jax.numpy as jnp
from jax import lax
from jax.experimental import pallas as pl
from jax.experimental.pallas import tpu as pltpu
from jax.experimental.pallas import tpu_sc as plsc

NC = 2
NS = 16
L = 16

P = 20
C = 10
ACC = 384
PCI_PAD = 224
CHUNK = 2048


def _sc_body(pix_per_w, chunks, pa_hbm, lab_hbm, pci_hbm, out_hbm,
             dbuf0, dbuf1, lbuf0, lbuf1, accs, accc, rbuf0, rbuf1, pcib,
             outv, cstage, sem_d0, sem_l0, sem_d1, sem_l1, shared):
    c = lax.axis_index("c")
    s = lax.axis_index("s")
    w = c * NS + s
    iota = lax.iota(jnp.int32, L)
    zf = jnp.zeros((L,), jnp.float32)
    ones = jnp.ones((L,), jnp.float32)

    for j in range(ACC // L):
        accs[pl.ds(j * L, L)] = zf
        accc[pl.ds(j * L, L)] = zf

    pps = [(iota + 16 * i) // P for i in range(5)]
    prs = [(iota + 16 * i) % P for i in range(5)]

    bufs = [(dbuf0, lbuf0, sem_d0, sem_l0), (dbuf1, lbuf1, sem_d1, sem_l1)]

    def start(k, db, lb, sd, sl):
        bp = w * pix_per_w + k * CHUNK
        cd = pltpu.async_copy(pa_hbm.at[pl.ds(bp, CHUNK)], db, sd)
        cl = pltpu.async_copy(lab_hbm.at[pl.ds(bp, CHUNK)], lb, sl)
        return cd, cl

    pend = start(0, *bufs[0])
    for k in range(chunks):
        db, lb, _, _ = bufs[k % 2]
        nxt = start(k + 1, *bufs[(k + 1) % 2]) if k + 1 < chunks else None
        pend[0].wait()
        pend[1].wait()

        @functools.partial(plsc.parallel_loop, 0, CHUNK * P // 80, unroll=4)
        def group(g):
            for i in range(5):
                pix = pps[i] + g * 4
                x = plsc.load_gather(db, [pix, prs[i]])
                lbl = plsc.load_gather(lb, [pix])
                dest = lbl * P + prs[i]
                plsc.addupdate_scatter(accs, [dest], jnp.abs(x))
                plsc.addupdate_scatter(accc, [dest], ones)

        pend = nxt

    pltpu.sync_copy(accs, shared.at[s, pl.ds(0, ACC)])
    cstage[pl.ds(0, L)] = plsc.load_gather(accc, [iota * P])
    pltpu.sync_copy(cstage, shared.at[s, pl.ds(ACC, L)])
    plsc.subcore_barrier()

    @pl.when(s == 0)
    def _finalize():
        pltpu.sync_copy(pci_hbm, pcib)
        lane_ok = iota < C
        ls = zf
        tot = zf
        for bl in range(NS // 2):
            pltpu.sync_copy(shared.at[2 * bl], rbuf0)
            pltpu.sync_copy(shared.at[2 * bl + 1], rbuf1)
            cnt = (plsc.load_gather(rbuf0, [ACC + 1 + iota]) +
                   plsc.load_gather(rbuf1, [ACC + 1 + iota]))
            t = zf
            npv = zf
            for p in range(P):
                m_raw = plsc.load_gather(pcib, [p * C + iota])
                m = jnp.where(m_raw != 0, 1.0, 0.0).astype(jnp.float32)
                sp = (plsc.load_gather(rbuf0, [(iota + 1) * P + p]) +
                      plsc.load_gather(rbuf1, [(iota + 1) * P + p]))
                t = t + m * sp
                npv = npv + m
            valid = (cnt > 0.0) & (npv > 0.0) & lane_ok
            denom = jnp.where(valid, cnt * npv, 1.0)
            ls = ls + jnp.where(valid, t / denom, 0.0)
            tot = tot + jnp.where(valid, 1.0, 0.0)
        outv[pl.ds(0, L)] = ls
        outv[pl.ds(L, L)] = tot
        pltpu.sync_copy(outv, out_hbm.at[c])


def _tc_block(pa_ref, lab_ref, out_ref):
    nb = pl.program_id(1)
    lbl = lab_ref[0, 0, :]
    nblk = lbl.shape[0]
    cls = jax.lax.broadcasted_iota(jnp.int32, (C, nblk), 0) + 1
    onehot_t = jnp.where(lbl[None, :] == cls, 1.0, 0.0).astype(jnp.float32)
    x = jnp.abs(pa_ref[0])
    xx = jnp.concatenate([x, jnp.ones((nblk, 1), jnp.float32)], axis=1)
    partial = jax.lax.dot_general(
        onehot_t, xx, (((1,), (0,)), ((), ())),
        preferred_element_type=jnp.float32)

    @pl.when(nb == 0)
    def _():
        out_ref[0] = jnp.zeros_like(out_ref[0])

    out_ref[0] += partial


def _tc_sums(pa, labels, block_n):
    B, N, _ = pa.shape
    lab3 = labels.reshape(B, 1, N)
    return pl.pallas_call(
        _tc_block,
        grid=(B, N // block_n),
        in_specs=[
            pl.BlockSpec((1, block_n, P), lambda b, n: (b, n, 0)),
            pl.BlockSpec((1, 1, block_n), lambda b, n: (b, 0, n)),
        ],
        out_specs=pl.BlockSpec((1, C, P + 1), lambda b, n: (b, 0, 0)),
        out_shape=jax.ShapeDtypeStruct((B, C, P + 1), jnp.float32),
        compiler_params=pltpu.CompilerParams(
            dimension_semantics=("parallel", "arbitrary")),
    )(pa, lab3)


def kernel(prototype_activations, target_labels, prototype_class_identity):
    B, N, Pp = prototype_activations.shape
    assert Pp == P and prototype_class_identity.shape == (P, C)
    out = _tc_sums(prototype_activations,
                   target_labels.astype(jnp.int32), 4096)
    sums = out[:, :, :P]
    counts = out[:, :, P]
    proto_mask = (prototype_class_identity != 0).astype(jnp.float32)
    n_protos = jnp.sum(proto_mask, axis=0)
    safe_counts = jnp.where(counts > 0, counts, 1.0)
    terms = jnp.einsum("bcp,pc->bc", sums / safe_counts[:, :, None],
                       proto_mask) / jnp.where(n_protos > 0, n_protos, 1.0)
    valid = ((counts > 0) & (n_protos[None, :] > 0)).astype(jnp.float32)
    total = jnp.sum(valid)
    loss_sum = jnp.sum(valid * terms)
    return jnp.where(total > 0,
                     loss_sum / jnp.where(total > 0, total, 1.0),
                     jnp.asarray(0.0, dtype=jnp.float32))


def _sc_kernel_unused(prototype_activations, target_labels, prototype_class_identity):
    B, N, Pp = prototype_activations.shape
    assert Pp == P and prototype_class_identity.shape == (P, C)
    n_pix = B * N
    pix_per_w = n_pix // (NC * NS)
    chunks = pix_per_w // CHUNK

    pa_flat = prototype_activations.reshape(n_pix, P)
    lab_flat = target_labels.reshape(-1).astype(jnp.int32)
    pci_pad = jnp.zeros((PCI_PAD,), jnp.float32).at[: P * C].set(
        prototype_class_identity.reshape(-1).astype(jnp.float32))

    mesh = plsc.VectorSubcoreMesh(
        core_axis_name="c", subcore_axis_name="s",
        num_cores=NC, num_subcores=NS)

    run = functools.partial(
        pl.kernel,
        out_type=jax.ShapeDtypeStruct((NC, 2 * L), jnp.float32),
        mesh=mesh,
        scratch_types=[
            pltpu.VMEM((CHUNK, P), jnp.float32),
            pltpu.VMEM((CHUNK, P), jnp.float32),
            pltpu.VMEM((CHUNK,), jnp.int32),
            pltpu.VMEM((CHUNK,), jnp.int32),
            pltpu.VMEM((ACC,), jnp.float32),
            pltpu.VMEM((ACC,), jnp.float32),
            pltpu.VMEM((512,), jnp.float32),
            pltpu.VMEM((512,), jnp.float32),
            pltpu.VMEM((PCI_PAD,), jnp.float32),
            pltpu.VMEM((2 * L,), jnp.float32),
            pltpu.VMEM((L,), jnp.float32),
            pltpu.SemaphoreType.DMA,
            pltpu.SemaphoreType.DMA,
            pltpu.SemaphoreType.DMA,
            pltpu.SemaphoreType.DMA,
            pltpu.VMEM_SHARED((NS, 512), jnp.float32),
        ],
        compiler_params=pltpu.CompilerParams(needs_layout_passes=False),
    )(functools.partial(_sc_body, pix_per_w, chunks))

    out = run(pa_flat, lab_flat, pci_pad)
    ls = jnp.sum(out[:, :L])
    tot = jnp.sum(out[:, L:])
    return jnp.where(tot > 0, ls / jnp.where(tot > 0, tot, 1.0),
                     jnp.asarray(0.0, dtype=jnp.float32))


# speedup vs baseline: 2.2278x; 1.1198x over previous
import functools

import jax
import jax.numpy as jnp
from jax import lax
from jax.experimental import pallas as pl
from jax.experimental.pallas import tpu as pltpu
from jax.experimental.pallas import tpu_sc as plsc

NC = 2
NS = 16
L = 16

P = 20
C = 10
ACC = 384
PCI_PAD = 224
CHUNK = 2048


def _sc_body(pix_per_w, chunks, pa_hbm, lab_hbm, pci_hbm, out_hbm,
             dbuf0, dbuf1, lbuf0, lbuf1, accs, accc, rbuf0, rbuf1, pcib,
             outv, cstage, sem_d0, sem_l0, sem_d1, sem_l1, shared):
    c = lax.axis_index("c")
    s = lax.axis_index("s")
    w = c * NS + s
    iota = lax.iota(jnp.int32, L)
    zf = jnp.zeros((L,), jnp.float32)
    ones = jnp.ones((L,), jnp.float32)

    for j in range(ACC // L):
        accs[pl.ds(j * L, L)] = zf
        accc[pl.ds(j * L, L)] = zf

    pps = [(iota + 16 * i) // P for i in range(5)]
    prs = [(iota + 16 * i) % P for i in range(5)]

    bufs = [(dbuf0, lbuf0, sem_d0, sem_l0), (dbuf1, lbuf1, sem_d1, sem_l1)]

    def start(k, db, lb, sd, sl):
        bp = w * pix_per_w + k * CHUNK
        cd = pltpu.async_copy(pa_hbm.at[pl.ds(bp, CHUNK)], db, sd)
        cl = pltpu.async_copy(lab_hbm.at[pl.ds(bp, CHUNK)], lb, sl)
        return cd, cl

    pend = start(0, *bufs[0])
    for k in range(chunks):
        db, lb, _, _ = bufs[k % 2]
        nxt = start(k + 1, *bufs[(k + 1) % 2]) if k + 1 < chunks else None
        pend[0].wait()
        pend[1].wait()

        @functools.partial(plsc.parallel_loop, 0, CHUNK * P // 80, unroll=4)
        def group(g):
            for i in range(5):
                pix = pps[i] + g * 4
                x = plsc.load_gather(db, [pix, prs[i]])
                lbl = plsc.load_gather(lb, [pix])
                dest = lbl * P + prs[i]
                plsc.addupdate_scatter(accs, [dest], jnp.abs(x))
                plsc.addupdate_scatter(accc, [dest], ones)

        pend = nxt

    pltpu.sync_copy(accs, shared.at[s, pl.ds(0, ACC)])
    cstage[pl.ds(0, L)] = plsc.load_gather(accc, [iota * P])
    pltpu.sync_copy(cstage, shared.at[s, pl.ds(ACC, L)])
    plsc.subcore_barrier()

    @pl.when(s == 0)
    def _finalize():
        pltpu.sync_copy(pci_hbm, pcib)
        lane_ok = iota < C
        ls = zf
        tot = zf
        for bl in range(NS // 2):
            pltpu.sync_copy(shared.at[2 * bl], rbuf0)
            pltpu.sync_copy(shared.at[2 * bl + 1], rbuf1)
            cnt = (plsc.load_gather(rbuf0, [ACC + 1 + iota]) +
                   plsc.load_gather(rbuf1, [ACC + 1 + iota]))
            t = zf
            npv = zf
            for p in range(P):
                m_raw = plsc.load_gather(pcib, [p * C + iota])
                m = jnp.where(m_raw != 0, 1.0, 0.0).astype(jnp.float32)
                sp = (plsc.load_gather(rbuf0, [(iota + 1) * P + p]) +
                      plsc.load_gather(rbuf1, [(iota + 1) * P + p]))
                t = t + m * sp
                npv = npv + m
            valid = (cnt > 0.0) & (npv > 0.0) & lane_ok
            denom = jnp.where(valid, cnt * npv, 1.0)
            ls = ls + jnp.where(valid, t / denom, 0.0)
            tot = tot + jnp.where(valid, 1.0, 0.0)
        outv[pl.ds(0, L)] = ls
        outv[pl.ds(L, L)] = tot
        pltpu.sync_copy(outv, out_hbm.at[c])


def _tc_block(pa_ref, lab_ref, out_ref):
    nb = pl.program_id(0)
    x = pa_ref[...]
    lbl = lab_ref[...]
    Bb, nblk = lbl.shape
    cls = jax.lax.broadcasted_iota(jnp.int32, (C, Bb, nblk), 0) + 1
    onehot = jnp.where(lbl[None, :, :] == cls, 1.0, 0.0).astype(jnp.float32)
    xx = jnp.concatenate(
        [jnp.abs(x), jnp.ones((1, Bb, nblk), jnp.float32)], axis=0)
    partial = jnp.einsum("cbn,pbn->bcp", onehot, xx,
                         preferred_element_type=jnp.float32)

    @pl.when(nb == 0)
    def _():
        out_ref[...] = jnp.zeros_like(out_ref)

    out_ref[...] += partial


def _tc_sums(pa, labels, block_n):
    B, N, _ = pa.shape
    pa_t = jnp.transpose(pa, (2, 0, 1))
    return pl.pallas_call(
        _tc_block,
        grid=(N // block_n,),
        in_specs=[
            pl.BlockSpec((P, B, block_n), lambda n: (0, 0, n)),
            pl.BlockSpec((B, block_n), lambda n: (0, n)),
        ],
        out_specs=pl.BlockSpec((B, C, P + 1), lambda n: (0, 0, 0)),
        out_shape=jax.ShapeDtypeStruct((B, C, P + 1), jnp.float32),
        compiler_params=pltpu.CompilerParams(
            dimension_semantics=("arbitrary",)),
    )(pa_t, labels)


def kernel(prototype_activations, target_labels, prototype_class_identity):
    B, N, Pp = prototype_activations.shape
    assert Pp == P and prototype_class_identity.shape == (P, C)
    out = _tc_sums(prototype_activations,
                   target_labels.astype(jnp.int32), 4096)
    sums = out[:, :, :P]
    counts = out[:, :, P]
    proto_mask = (prototype_class_identity != 0).astype(jnp.float32)
    n_protos = jnp.sum(proto_mask, axis=0)
    safe_counts = jnp.where(counts > 0, counts, 1.0)
    terms = jnp.einsum("bcp,pc->bc", sums / safe_counts[:, :, None],
                       proto_mask) / jnp.where(n_protos > 0, n_protos, 1.0)
    valid = ((counts > 0) & (n_protos[None, :] > 0)).astype(jnp.float32)
    total = jnp.sum(valid)
    loss_sum = jnp.sum(valid * terms)
    return jnp.where(total > 0,
                     loss_sum / jnp.where(total > 0, total, 1.0),
                     jnp.asarray(0.0, dtype=jnp.float32))


def _sc_kernel_unused(prototype_activations, target_labels, prototype_class_identity):
    B, N, Pp = prototype_activations.shape
    assert Pp == P and prototype_class_identity.shape == (P, C)
    n_pix = B * N
    pix_per_w = n_pix // (NC * NS)
    chunks = pix_per_w // CHUNK

    pa_flat = prototype_activations.reshape(n_pix, P)
    lab_flat = target_labels.reshape(-1).astype(jnp.int32)
    pci_pad = jnp.zeros((PCI_PAD,), jnp.float32).at[: P * C].set(
        prototype_class_identity.reshape(-1).astype(jnp.float32))

    mesh = plsc.VectorSubcoreMesh(
        core_axis_name="c", subcore_axis_name="s",
        num_cores=NC, num_subcores=NS)

    run = functools.partial(
        pl.kernel,
        out_type=jax.ShapeDtypeStruct((NC, 2 * L), jnp.float32),
        mesh=mesh,
        scratch_types=[
            pltpu.VMEM((CHUNK, P), jnp.float32),
            pltpu.VMEM((CHUNK, P), jnp.float32),
            pltpu.VMEM((CHUNK,), jnp.int32),
            pltpu.VMEM((CHUNK,), jnp.int32),
            pltpu.VMEM((ACC,), jnp.float32),
            pltpu.VMEM((ACC,), jnp.float32),
            pltpu.VMEM((512,), jnp.float32),
            pltpu.VMEM((512,), jnp.float32),
            pltpu.VMEM((PCI_PAD,), jnp.float32),
            pltpu.VMEM((2 * L,), jnp.float32),
            pltpu.VMEM((L,), jnp.float32),
            pltpu.SemaphoreType.DMA,
            pltpu.SemaphoreType.DMA,
            pltpu.SemaphoreType.DMA,
            pltpu.SemaphoreType.DMA,
            pltpu.VMEM_SHARED((NS, 512), jnp.float32),
        ],
        compiler_params=pltpu.CompilerParams(needs_layout_passes=False),
    )(functools.partial(_sc_body, pix_per_w, chunks))

    out = run(pa_flat, lab_flat, pci_pad)
    ls = jnp.sum(out[:, :L])
    tot = jnp.sum(out[:, L:])
    return jnp.where(tot > 0, ls / jnp.where(tot > 0, tot, 1.0),
                     jnp.asarray(0.0, dtype=jnp.float32))


# speedup vs baseline: 5.5717x; 2.5010x over previous
import functools

import jax
import jax.numpy as jnp
from jax import lax
from jax.experimental import pallas as pl
from jax.experimental.pallas import tpu as pltpu
from jax.experimental.pallas import tpu_sc as plsc

NC = 2
NS = 16
L = 16

P = 20
C = 10
ACC = 384
PCI_PAD = 224
CHUNK = 2048


def _sc_body(pix_per_w, chunks, pa_hbm, lab_hbm, pci_hbm, out_hbm,
             dbuf0, dbuf1, lbuf0, lbuf1, accs, accc, rbuf0, rbuf1, pcib,
             outv, cstage, sem_d0, sem_l0, sem_d1, sem_l1, shared):
    c = lax.axis_index("c")
    s = lax.axis_index("s")
    w = c * NS + s
    iota = lax.iota(jnp.int32, L)
    zf = jnp.zeros((L,), jnp.float32)
    ones = jnp.ones((L,), jnp.float32)

    for j in range(ACC // L):
        accs[pl.ds(j * L, L)] = zf
        accc[pl.ds(j * L, L)] = zf

    pps = [(iota + 16 * i) // P for i in range(5)]
    prs = [(iota + 16 * i) % P for i in range(5)]

    bufs = [(dbuf0, lbuf0, sem_d0, sem_l0), (dbuf1, lbuf1, sem_d1, sem_l1)]

    def start(k, db, lb, sd, sl):
        bp = w * pix_per_w + k * CHUNK
        cd = pltpu.async_copy(pa_hbm.at[pl.ds(bp, CHUNK)], db, sd)
        cl = pltpu.async_copy(lab_hbm.at[pl.ds(bp, CHUNK)], lb, sl)
        return cd, cl

    pend = start(0, *bufs[0])
    for k in range(chunks):
        db, lb, _, _ = bufs[k % 2]
        nxt = start(k + 1, *bufs[(k + 1) % 2]) if k + 1 < chunks else None
        pend[0].wait()
        pend[1].wait()

        @functools.partial(plsc.parallel_loop, 0, CHUNK * P // 80, unroll=4)
        def group(g):
            for i in range(5):
                pix = pps[i] + g * 4
                x = plsc.load_gather(db, [pix, prs[i]])
                lbl = plsc.load_gather(lb, [pix])
                dest = lbl * P + prs[i]
                plsc.addupdate_scatter(accs, [dest], jnp.abs(x))
                plsc.addupdate_scatter(accc, [dest], ones)

        pend = nxt

    pltpu.sync_copy(accs, shared.at[s, pl.ds(0, ACC)])
    cstage[pl.ds(0, L)] = plsc.load_gather(accc, [iota * P])
    pltpu.sync_copy(cstage, shared.at[s, pl.ds(ACC, L)])
    plsc.subcore_barrier()

    @pl.when(s == 0)
    def _finalize():
        pltpu.sync_copy(pci_hbm, pcib)
        lane_ok = iota < C
        ls = zf
        tot = zf
        for bl in range(NS // 2):
            pltpu.sync_copy(shared.at[2 * bl], rbuf0)
            pltpu.sync_copy(shared.at[2 * bl + 1], rbuf1)
            cnt = (plsc.load_gather(rbuf0, [ACC + 1 + iota]) +
                   plsc.load_gather(rbuf1, [ACC + 1 + iota]))
            t = zf
            npv = zf
            for p in range(P):
                m_raw = plsc.load_gather(pcib, [p * C + iota])
                m = jnp.where(m_raw != 0, 1.0, 0.0).astype(jnp.float32)
                sp = (plsc.load_gather(rbuf0, [(iota + 1) * P + p]) +
                      plsc.load_gather(rbuf1, [(iota + 1) * P + p]))
                t = t + m * sp
                npv = npv + m
            valid = (cnt > 0.0) & (npv > 0.0) & lane_ok
            denom = jnp.where(valid, cnt * npv, 1.0)
            ls = ls + jnp.where(valid, t / denom, 0.0)
            tot = tot + jnp.where(valid, 1.0, 0.0)
        outv[pl.ds(0, L)] = ls
        outv[pl.ds(L, L)] = tot
        pltpu.sync_copy(outv, out_hbm.at[c])


G = 8
GC = G * C
GP = G * P


def _tc_block(pa_ref, lab_ref, out_ref):
    nb = pl.program_id(1)
    nblk = pa_ref.shape[2]
    x2 = jnp.abs(pa_ref[...]).reshape(GP, nblk)
    lbl = lab_ref[...]
    lt = jnp.transpose(lbl)
    ltr = jnp.concatenate(
        [jnp.broadcast_to(lt[:, b:b + 1], (nblk, C)) for b in range(G)],
        axis=1)
    cls = jax.lax.broadcasted_iota(jnp.int32, (nblk, GC), 1) % C + 1
    oh = jnp.where(ltr == cls, 1.0, 0.0).astype(jnp.float32)
    partial = jax.lax.dot_general(
        x2, oh, (((1,), (0,)), ((), ())),
        preferred_element_type=jnp.float32)
    counts = jnp.sum(oh, axis=0)

    @pl.when(nb == 0)
    def _():
        out_ref[...] = jnp.zeros_like(out_ref)

    out_ref[0, :GP, :] += partial
    out_ref[0, GP, :] += counts


def _tc_sums(pa, labels, block_n):
    B, N, _ = pa.shape
    pa_t = jnp.transpose(pa, (2, 0, 1))
    return pl.pallas_call(
        _tc_block,
        grid=(B // G, N // block_n),
        in_specs=[
            pl.BlockSpec((P, G, block_n), lambda h, n: (0, h, n)),
            pl.BlockSpec((G, block_n), lambda h, n: (h, n)),
        ],
        out_specs=pl.BlockSpec((1, GP + 1, GC), lambda h, n: (h, 0, 0)),
        out_shape=jax.ShapeDtypeStruct((B // G, GP + 1, GC), jnp.float32),
        compiler_params=pltpu.CompilerParams(
            dimension_semantics=("parallel", "arbitrary")),
    )(pa_t, labels)


def kernel(prototype_activations, target_labels, prototype_class_identity):
    B, N, Pp = prototype_activations.shape
    assert Pp == P and prototype_class_identity.shape == (P, C)
    out = _tc_sums(prototype_activations,
                   target_labels.astype(jnp.int32), 8192)
    o5 = out[:, :GP, :].reshape(B // G, P, G, G, C)
    sums = jnp.einsum("hpbbc->hbpc", o5).reshape(B, P, C)
    counts = out[:, GP, :].reshape(B, C)
    proto_mask = (prototype_class_identity != 0).astype(jnp.float32)
    n_protos = jnp.sum(proto_mask, axis=0)
    safe_counts = jnp.where(counts > 0, counts, 1.0)
    terms = jnp.einsum("bpc,pc->bc", sums / safe_counts[:, None, :],
                       proto_mask) / jnp.where(n_protos > 0, n_protos, 1.0)
    valid = ((counts > 0) & (n_protos[None, :] > 0)).astype(jnp.float32)
    total = jnp.sum(valid)
    loss_sum = jnp.sum(valid * terms)
    return jnp.where(total > 0,
                     loss_sum / jnp.where(total > 0, total, 1.0),
                     jnp.asarray(0.0, dtype=jnp.float32))


def _sc_kernel_unused(prototype_activations, target_labels, prototype_class_identity):
    B, N, Pp = prototype_activations.shape
    assert Pp == P and prototype_class_identity.shape == (P, C)
    n_pix = B * N
    pix_per_w = n_pix // (NC * NS)
    chunks = pix_per_w // CHUNK

    pa_flat = prototype_activations.reshape(n_pix, P)
    lab_flat = target_labels.reshape(-1).astype(jnp.int32)
    pci_pad = jnp.zeros((PCI_PAD,), jnp.float32).at[: P * C].set(
        prototype_class_identity.reshape(-1).astype(jnp.float32))

    mesh = plsc.VectorSubcoreMesh(
        core_axis_name="c", subcore_axis_name="s",
        num_cores=NC, num_subcores=NS)

    run = functools.partial(
        pl.kernel,
        out_type=jax.ShapeDtypeStruct((NC, 2 * L), jnp.float32),
        mesh=mesh,
        scratch_types=[
            pltpu.VMEM((CHUNK, P), jnp.float32),
            pltpu.VMEM((CHUNK, P), jnp.float32),
            pltpu.VMEM((CHUNK,), jnp.int32),
            pltpu.VMEM((CHUNK,), jnp.int32),
            pltpu.VMEM((ACC,), jnp.float32),
            pltpu.VMEM((ACC,), jnp.float32),
            pltpu.VMEM((512,), jnp.float32),
            pltpu.VMEM((512,), jnp.float32),
            pltpu.VMEM((PCI_PAD,), jnp.float32),
            pltpu.VMEM((2 * L,), jnp.float32),
            pltpu.VMEM((L,), jnp.float32),
            pltpu.SemaphoreType.DMA,
            pltpu.SemaphoreType.DMA,
            pltpu.SemaphoreType.DMA,
            pltpu.SemaphoreType.DMA,
            pltpu.VMEM_SHARED((NS, 512), jnp.float32),
        ],
        compiler_params=pltpu.CompilerParams(needs_layout_passes=False),
    )(functools.partial(_sc_body, pix_per_w, chunks))

    out = run(pa_flat, lab_flat, pci_pad)
    ls = jnp.sum(out[:, :L])
    tot = jnp.sum(out[:, L:])
    return jnp.where(tot > 0, ls / jnp.where(tot > 0, tot, 1.0),
                     jnp.asarray(0.0, dtype=jnp.float32))


# speedup vs baseline: 24.2387x; 4.3503x over previous
import functools

import jax
import jax.numpy as jnp
from jax import lax
from jax.experimental import pallas as pl
from jax.experimental.pallas import tpu as pltpu
from jax.experimental.pallas import tpu_sc as plsc

NC = 2
NS = 16
L = 16

P = 20
C = 10
ACC = 384
PCI_PAD = 224
CHUNK = 2048


def _sc_body(pix_per_w, chunks, pa_hbm, lab_hbm, pci_hbm, out_hbm,
             dbuf0, dbuf1, lbuf0, lbuf1, accs, accc, rbuf0, rbuf1, pcib,
             outv, cstage, sem_d0, sem_l0, sem_d1, sem_l1, shared):
    c = lax.axis_index("c")
    s = lax.axis_index("s")
    w = c * NS + s
    iota = lax.iota(jnp.int32, L)
    zf = jnp.zeros((L,), jnp.float32)
    ones = jnp.ones((L,), jnp.float32)

    for j in range(ACC // L):
        accs[pl.ds(j * L, L)] = zf
        accc[pl.ds(j * L, L)] = zf

    pps = [(iota + 16 * i) // P for i in range(5)]
    prs = [(iota + 16 * i) % P for i in range(5)]

    bufs = [(dbuf0, lbuf0, sem_d0, sem_l0), (dbuf1, lbuf1, sem_d1, sem_l1)]

    def start(k, db, lb, sd, sl):
        bp = w * pix_per_w + k * CHUNK
        cd = pltpu.async_copy(pa_hbm.at[pl.ds(bp, CHUNK)], db, sd)
        cl = pltpu.async_copy(lab_hbm.at[pl.ds(bp, CHUNK)], lb, sl)
        return cd, cl

    pend = start(0, *bufs[0])
    for k in range(chunks):
        db, lb, _, _ = bufs[k % 2]
        nxt = start(k + 1, *bufs[(k + 1) % 2]) if k + 1 < chunks else None
        pend[0].wait()
        pend[1].wait()

        @functools.partial(plsc.parallel_loop, 0, CHUNK * P // 80, unroll=4)
        def group(g):
            for i in range(5):
                pix = pps[i] + g * 4
                x = plsc.load_gather(db, [pix, prs[i]])
                lbl = plsc.load_gather(lb, [pix])
                dest = lbl * P + prs[i]
                plsc.addupdate_scatter(accs, [dest], jnp.abs(x))
                plsc.addupdate_scatter(accc, [dest], ones)

        pend = nxt

    pltpu.sync_copy(accs, shared.at[s, pl.ds(0, ACC)])
    cstage[pl.ds(0, L)] = plsc.load_gather(accc, [iota * P])
    pltpu.sync_copy(cstage, shared.at[s, pl.ds(ACC, L)])
    plsc.subcore_barrier()

    @pl.when(s == 0)
    def _finalize():
        pltpu.sync_copy(pci_hbm, pcib)
        lane_ok = iota < C
        ls = zf
        tot = zf
        for bl in range(NS // 2):
            pltpu.sync_copy(shared.at[2 * bl], rbuf0)
            pltpu.sync_copy(shared.at[2 * bl + 1], rbuf1)
            cnt = (plsc.load_gather(rbuf0, [ACC + 1 + iota]) +
                   plsc.load_gather(rbuf1, [ACC + 1 + iota]))
            t = zf
            npv = zf
            for p in range(P):
                m_raw = plsc.load_gather(pcib, [p * C + iota])
                m = jnp.where(m_raw != 0, 1.0, 0.0).astype(jnp.float32)
                sp = (plsc.load_gather(rbuf0, [(iota + 1) * P + p]) +
                      plsc.load_gather(rbuf1, [(iota + 1) * P + p]))
                t = t + m * sp
                npv = npv + m
            valid = (cnt > 0.0) & (npv > 0.0) & lane_ok
            denom = jnp.where(valid, cnt * npv, 1.0)
            ls = ls + jnp.where(valid, t / denom, 0.0)
            tot = tot + jnp.where(valid, 1.0, 0.0)
        outv[pl.ds(0, L)] = ls
        outv[pl.ds(L, L)] = tot
        pltpu.sync_copy(outv, out_hbm.at[c])


G = 8
GC = G * C
GP = G * P


def _tc_block(pa_ref, lab_ref, out_ref):
    nb = pl.program_id(1)
    nblk = pa_ref.shape[2]
    x2 = jnp.abs(pa_ref[...]).reshape(GP, nblk)
    lbl = lab_ref[...]
    lt = jnp.transpose(lbl).astype(jnp.float32)
    exp_rows = jax.lax.broadcasted_iota(jnp.int32, (G, GC), 0)
    exp_cols = jax.lax.broadcasted_iota(jnp.int32, (G, GC), 1) // C
    expand = jnp.where(exp_rows == exp_cols, 1.0, 0.0).astype(jnp.float32)
    ltr = jax.lax.dot_general(
        lt, expand, (((1,), (0,)), ((), ())),
        preferred_element_type=jnp.float32)
    cls = (jax.lax.broadcasted_iota(jnp.int32, (nblk, GC), 1) % C
           + 1).astype(jnp.float32)
    oh = jnp.where(ltr == cls, 1.0, 0.0).astype(jnp.float32)
    partial = jax.lax.dot_general(
        x2, oh, (((1,), (0,)), ((), ())),
        preferred_element_type=jnp.float32)
    counts = jnp.sum(oh, axis=0)

    @pl.when(nb == 0)
    def _():
        out_ref[...] = jnp.zeros_like(out_ref)

    out_ref[0, :GP, :] += partial
    out_ref[0, GP, :] += counts


def _tc_sums(pa, labels, block_n):
    B, N, _ = pa.shape
    pa_t = jnp.transpose(pa, (2, 0, 1))
    return pl.pallas_call(
        _tc_block,
        grid=(B // G, N // block_n),
        in_specs=[
            pl.BlockSpec((P, G, block_n), lambda h, n: (0, h, n)),
            pl.BlockSpec((G, block_n), lambda h, n: (h, n)),
        ],
        out_specs=pl.BlockSpec((1, GP + 1, GC), lambda h, n: (h, 0, 0)),
        out_shape=jax.ShapeDtypeStruct((B // G, GP + 1, GC), jnp.float32),
        compiler_params=pltpu.CompilerParams(
            dimension_semantics=("parallel", "arbitrary")),
    )(pa_t, labels)


def kernel(prototype_activations, target_labels, prototype_class_identity):
    B, N, Pp = prototype_activations.shape
    assert Pp == P and prototype_class_identity.shape == (P, C)
    out = _tc_sums(prototype_activations,
                   target_labels.astype(jnp.int32), 8192)
    o5 = out[:, :GP, :].reshape(B // G, P, G, G, C)
    sums = jnp.einsum("hpbbc->hbpc", o5).reshape(B, P, C)
    counts = out[:, GP, :].reshape(B, C)
    proto_mask = (prototype_class_identity != 0).astype(jnp.float32)
    n_protos = jnp.sum(proto_mask, axis=0)
    safe_counts = jnp.where(counts > 0, counts, 1.0)
    terms = jnp.einsum("bpc,pc->bc", sums / safe_counts[:, None, :],
                       proto_mask) / jnp.where(n_protos > 0, n_protos, 1.0)
    valid = ((counts > 0) & (n_protos[None, :] > 0)).astype(jnp.float32)
    total = jnp.sum(valid)
    loss_sum = jnp.sum(valid * terms)
    return jnp.where(total > 0,
                     loss_sum / jnp.where(total > 0, total, 1.0),
                     jnp.asarray(0.0, dtype=jnp.float32))


def _sc_kernel_unused(prototype_activations, target_labels, prototype_class_identity):
    B, N, Pp = prototype_activations.shape
    assert Pp == P and prototype_class_identity.shape == (P, C)
    n_pix = B * N
    pix_per_w = n_pix // (NC * NS)
    chunks = pix_per_w // CHUNK

    pa_flat = prototype_activations.reshape(n_pix, P)
    lab_flat = target_labels.reshape(-1).astype(jnp.int32)
    pci_pad = jnp.zeros((PCI_PAD,), jnp.float32).at[: P * C].set(
        prototype_class_identity.reshape(-1).astype(jnp.float32))

    mesh = plsc.VectorSubcoreMesh(
        core_axis_name="c", subcore_axis_name="s",
        num_cores=NC, num_subcores=NS)

    run = functools.partial(
        pl.kernel,
        out_type=jax.ShapeDtypeStruct((NC, 2 * L), jnp.float32),
        mesh=mesh,
        scratch_types=[
            pltpu.VMEM((CHUNK, P), jnp.float32),
            pltpu.VMEM((CHUNK, P), jnp.float32),
            pltpu.VMEM((CHUNK,), jnp.int32),
            pltpu.VMEM((CHUNK,), jnp.int32),
            pltpu.VMEM((ACC,), jnp.float32),
            pltpu.VMEM((ACC,), jnp.float32),
            pltpu.VMEM((512,), jnp.float32),
            pltpu.VMEM((512,), jnp.float32),
            pltpu.VMEM((PCI_PAD,), jnp.float32),
            pltpu.VMEM((2 * L,), jnp.float32),
            pltpu.VMEM((L,), jnp.float32),
            pltpu.SemaphoreType.DMA,
            pltpu.SemaphoreType.DMA,
            pltpu.SemaphoreType.DMA,
            pltpu.SemaphoreType.DMA,
            pltpu.VMEM_SHARED((NS, 512), jnp.float32),
        ],
        compiler_params=pltpu.CompilerParams(needs_layout_passes=False),
    )(functools.partial(_sc_body, pix_per_w, chunks))

    out = run(pa_flat, lab_flat, pci_pad)
    ls = jnp.sum(out[:, :L])
    tot = jnp.sum(out[:, L:])
    return jnp.where(tot > 0, ls / jnp.where(tot > 0, tot, 1.0),
                     jnp.asarray(0.0, dtype=jnp.float32))


# speedup vs baseline: 26.5533x; 1.0955x over previous
import functools

import jax
import jax.numpy as jnp
from jax import lax
from jax.experimental import pallas as pl
from jax.experimental.pallas import tpu as pltpu
from jax.experimental.pallas import tpu_sc as plsc

NC = 2
NS = 16
L = 16

P = 20
C = 10
ACC = 384
PCI_PAD = 224
CHUNK = 2048


def _sc_body(pix_per_w, chunks, pa_hbm, lab_hbm, pci_hbm, out_hbm,
             dbuf0, dbuf1, lbuf0, lbuf1, accs, accc, rbuf0, rbuf1, pcib,
             outv, cstage, sem_d0, sem_l0, sem_d1, sem_l1, shared):
    c = lax.axis_index("c")
    s = lax.axis_index("s")
    w = c * NS + s
    iota = lax.iota(jnp.int32, L)
    zf = jnp.zeros((L,), jnp.float32)
    ones = jnp.ones((L,), jnp.float32)

    for j in range(ACC // L):
        accs[pl.ds(j * L, L)] = zf
        accc[pl.ds(j * L, L)] = zf

    pps = [(iota + 16 * i) // P for i in range(5)]
    prs = [(iota + 16 * i) % P for i in range(5)]

    bufs = [(dbuf0, lbuf0, sem_d0, sem_l0), (dbuf1, lbuf1, sem_d1, sem_l1)]

    def start(k, db, lb, sd, sl):
        bp = w * pix_per_w + k * CHUNK
        cd = pltpu.async_copy(pa_hbm.at[pl.ds(bp, CHUNK)], db, sd)
        cl = pltpu.async_copy(lab_hbm.at[pl.ds(bp, CHUNK)], lb, sl)
        return cd, cl

    pend = start(0, *bufs[0])
    for k in range(chunks):
        db, lb, _, _ = bufs[k % 2]
        nxt = start(k + 1, *bufs[(k + 1) % 2]) if k + 1 < chunks else None
        pend[0].wait()
        pend[1].wait()

        @functools.partial(plsc.parallel_loop, 0, CHUNK * P // 80, unroll=4)
        def group(g):
            for i in range(5):
                pix = pps[i] + g * 4
                x = plsc.load_gather(db, [pix, prs[i]])
                lbl = plsc.load_gather(lb, [pix])
                dest = lbl * P + prs[i]
                plsc.addupdate_scatter(accs, [dest], jnp.abs(x))
                plsc.addupdate_scatter(accc, [dest], ones)

        pend = nxt

    pltpu.sync_copy(accs, shared.at[s, pl.ds(0, ACC)])
    cstage[pl.ds(0, L)] = plsc.load_gather(accc, [iota * P])
    pltpu.sync_copy(cstage, shared.at[s, pl.ds(ACC, L)])
    plsc.subcore_barrier()

    @pl.when(s == 0)
    def _finalize():
        pltpu.sync_copy(pci_hbm, pcib)
        lane_ok = iota < C
        ls = zf
        tot = zf
        for bl in range(NS // 2):
            pltpu.sync_copy(shared.at[2 * bl], rbuf0)
            pltpu.sync_copy(shared.at[2 * bl + 1], rbuf1)
            cnt = (plsc.load_gather(rbuf0, [ACC + 1 + iota]) +
                   plsc.load_gather(rbuf1, [ACC + 1 + iota]))
            t = zf
            npv = zf
            for p in range(P):
                m_raw = plsc.load_gather(pcib, [p * C + iota])
                m = jnp.where(m_raw != 0, 1.0, 0.0).astype(jnp.float32)
                sp = (plsc.load_gather(rbuf0, [(iota + 1) * P + p]) +
                      plsc.load_gather(rbuf1, [(iota + 1) * P + p]))
                t = t + m * sp
                npv = npv + m
            valid = (cnt > 0.0) & (npv > 0.0) & lane_ok
            denom = jnp.where(valid, cnt * npv, 1.0)
            ls = ls + jnp.where(valid, t / denom, 0.0)
            tot = tot + jnp.where(valid, 1.0, 0.0)
        outv[pl.ds(0, L)] = ls
        outv[pl.ds(L, L)] = tot
        pltpu.sync_copy(outv, out_hbm.at[c])


G = 8
GC = G * C
GP = G * P


def _tc_block(pa_ref, lab_ref, out_ref):
    nb = pl.program_id(1)
    nblk = pa_ref.shape[2]
    x2 = jnp.abs(pa_ref[...]).reshape(GP, nblk)
    lbl = lab_ref[...]
    lt = jnp.transpose(lbl).astype(jnp.float32)
    exp_rows = jax.lax.broadcasted_iota(jnp.int32, (G, GC), 0)
    exp_cols = jax.lax.broadcasted_iota(jnp.int32, (G, GC), 1) // C
    expand = jnp.where(exp_rows == exp_cols, 1.0, 0.0).astype(jnp.float32)
    ltr = jax.lax.dot_general(
        lt, expand, (((1,), (0,)), ((), ())),
        preferred_element_type=jnp.float32)
    cls = (jax.lax.broadcasted_iota(jnp.int32, (nblk, GC), 1) % C
           + 1).astype(jnp.float32)
    oh = jnp.where(ltr == cls, 1.0, 0.0).astype(jnp.float32)
    partial = jax.lax.dot_general(
        x2, oh, (((1,), (0,)), ((), ())),
        preferred_element_type=jnp.float32)
    counts = jnp.sum(oh, axis=0)

    @pl.when(nb == 0)
    def _():
        out_ref[...] = jnp.zeros_like(out_ref)

    out_ref[0, :GP, :] += partial
    out_ref[0, GP, :] += counts


def _tc_sums(pa, labels, block_n):
    B, N, _ = pa.shape
    pa_t = jnp.transpose(pa, (2, 0, 1))
    return pl.pallas_call(
        _tc_block,
        grid=(B // G, N // block_n),
        in_specs=[
            pl.BlockSpec((P, G, block_n), lambda h, n: (0, h, n)),
            pl.BlockSpec((G, block_n), lambda h, n: (h, n)),
        ],
        out_specs=pl.BlockSpec((1, GP + 1, GC), lambda h, n: (h, 0, 0)),
        out_shape=jax.ShapeDtypeStruct((B // G, GP + 1, GC), jnp.float32),
        compiler_params=pltpu.CompilerParams(
            dimension_semantics=("parallel", "arbitrary")),
    )(pa_t, labels)


def kernel(prototype_activations, target_labels, prototype_class_identity):
    B, N, Pp = prototype_activations.shape
    assert Pp == P and prototype_class_identity.shape == (P, C)
    out = _tc_sums(prototype_activations,
                   target_labels.astype(jnp.int32), 16384)
    o5 = out[:, :GP, :].reshape(B // G, P, G, G, C)
    sums = jnp.einsum("hpbbc->hbpc", o5).reshape(B, P, C)
    counts = out[:, GP, :].reshape(B, C)
    proto_mask = (prototype_class_identity != 0).astype(jnp.float32)
    n_protos = jnp.sum(proto_mask, axis=0)
    safe_counts = jnp.where(counts > 0, counts, 1.0)
    terms = jnp.einsum("bpc,pc->bc", sums / safe_counts[:, None, :],
                       proto_mask) / jnp.where(n_protos > 0, n_protos, 1.0)
    valid = ((counts > 0) & (n_protos[None, :] > 0)).astype(jnp.float32)
    total = jnp.sum(valid)
    loss_sum = jnp.sum(valid * terms)
    return jnp.where(total > 0,
                     loss_sum / jnp.where(total > 0, total, 1.0),
                     jnp.asarray(0.0, dtype=jnp.float32))


def _sc_kernel_unused(prototype_activations, target_labels, prototype_class_identity):
    B, N, Pp = prototype_activations.shape
    assert Pp == P and prototype_class_identity.shape == (P, C)
    n_pix = B * N
    pix_per_w = n_pix // (NC * NS)
    chunks = pix_per_w // CHUNK

    pa_flat = prototype_activations.reshape(n_pix, P)
    lab_flat = target_labels.reshape(-1).astype(jnp.int32)
    pci_pad = jnp.zeros((PCI_PAD,), jnp.float32).at[: P * C].set(
        prototype_class_identity.reshape(-1).astype(jnp.float32))

    mesh = plsc.VectorSubcoreMesh(
        core_axis_name="c", subcore_axis_name="s",
        num_cores=NC, num_subcores=NS)

    run = functools.partial(
        pl.kernel,
        out_type=jax.ShapeDtypeStruct((NC, 2 * L), jnp.float32),
        mesh=mesh,
        scratch_types=[
            pltpu.VMEM((CHUNK, P), jnp.float32),
            pltpu.VMEM((CHUNK, P), jnp.float32),
            pltpu.VMEM((CHUNK,), jnp.int32),
            pltpu.VMEM((CHUNK,), jnp.int32),
            pltpu.VMEM((ACC,), jnp.float32),
            pltpu.VMEM((ACC,), jnp.float32),
            pltpu.VMEM((512,), jnp.float32),
            pltpu.VMEM((512,), jnp.float32),
            pltpu.VMEM((PCI_PAD,), jnp.float32),
            pltpu.VMEM((2 * L,), jnp.float32),
            pltpu.VMEM((L,), jnp.float32),
            pltpu.SemaphoreType.DMA,
            pltpu.SemaphoreType.DMA,
            pltpu.SemaphoreType.DMA,
            pltpu.SemaphoreType.DMA,
            pltpu.VMEM_SHARED((NS, 512), jnp.float32),
        ],
        compiler_params=pltpu.CompilerParams(needs_layout_passes=False),
    )(functools.partial(_sc_body, pix_per_w, chunks))

    out = run(pa_flat, lab_flat, pci_pad)
    ls = jnp.sum(out[:, :L])
    tot = jnp.sum(out[:, L:])
    return jnp.where(tot > 0, ls / jnp.where(tot > 0, tot, 1.0),
                     jnp.asarray(0.0, dtype=jnp.float32))


# speedup vs baseline: 26.5910x; 1.0014x over previous
import jax
import jax.numpy as jnp
from jax.experimental import pallas as pl
from jax.experimental.pallas import tpu as pltpu

P = 20
C = 10
G = 8
GC = G * C
GP = G * P
BLOCK_N = 16384


def _tc_block(pa_ref, lab_ref, out_ref):
    nb = pl.program_id(1)
    nblk = pa_ref.shape[2]
    x2 = jnp.abs(pa_ref[...]).reshape(GP, nblk)
    lbl = lab_ref[...]
    lt = jnp.transpose(lbl).astype(jnp.float32)
    exp_rows = jax.lax.broadcasted_iota(jnp.int32, (G, GC), 0)
    exp_cols = jax.lax.broadcasted_iota(jnp.int32, (G, GC), 1) // C
    expand = jnp.where(exp_rows == exp_cols, 1.0, 0.0).astype(jnp.float32)
    ltr = jax.lax.dot_general(
        lt, expand, (((1,), (0,)), ((), ())),
        preferred_element_type=jnp.float32)
    cls = (jax.lax.broadcasted_iota(jnp.int32, (nblk, GC), 1) % C
           + 1).astype(jnp.float32)
    oh = jnp.where(ltr == cls, 1.0, 0.0).astype(jnp.float32)
    partial = jax.lax.dot_general(
        x2, oh, (((1,), (0,)), ((), ())),
        preferred_element_type=jnp.float32)
    counts = jnp.sum(oh, axis=0)

    @pl.when(nb == 0)
    def _():
        out_ref[...] = jnp.zeros_like(out_ref)

    out_ref[0, :GP, :] += partial
    out_ref[0, GP, :] += counts


def _tc_sums(pa, labels, block_n):
    B, N, _ = pa.shape
    pa_t = jnp.transpose(pa, (2, 0, 1))
    return pl.pallas_call(
        _tc_block,
        grid=(B // G, N // block_n),
        in_specs=[
            pl.BlockSpec((P, G, block_n), lambda h, n: (0, h, n)),
            pl.BlockSpec((G, block_n), lambda h, n: (h, n)),
        ],
        out_specs=pl.BlockSpec((1, GP + 1, GC), lambda h, n: (h, 0, 0)),
        out_shape=jax.ShapeDtypeStruct((B // G, GP + 1, GC), jnp.float32),
        compiler_params=pltpu.CompilerParams(
            dimension_semantics=("parallel", "arbitrary")),
    )(pa_t, labels)


def kernel(prototype_activations, target_labels, prototype_class_identity):
    B, N, Pp = prototype_activations.shape
    assert Pp == P and prototype_class_identity.shape == (P, C)
    out = _tc_sums(prototype_activations,
                   target_labels.astype(jnp.int32), BLOCK_N)
    o5 = out[:, :GP, :].reshape(B // G, P, G, G, C)
    sums = jnp.einsum("hpbbc->hbpc", o5).reshape(B, P, C)
    counts = out[:, GP, :].reshape(B, C)
    proto_mask = (prototype_class_identity != 0).astype(jnp.float32)
    n_protos = jnp.sum(proto_mask, axis=0)
    safe_counts = jnp.where(counts > 0, counts, 1.0)
    terms = jnp.einsum("bpc,pc->bc", sums / safe_counts[:, None, :],
                       proto_mask) / jnp.where(n_protos > 0, n_protos, 1.0)
    valid = ((counts > 0) & (n_protos[None, :] > 0)).astype(jnp.float32)
    total = jnp.sum(valid)
    loss_sum = jnp.sum(valid * terms)
    return jnp.where(total > 0,
                     loss_sum / jnp.where(total > 0, total, 1.0),
                     jnp.asarray(0.0, dtype=jnp.float32))
